# baseline (speedup 1.0000x reference)
import functools
import math

import jax
import jax.numpy as jnp
from jax import lax
from jax.experimental import pallas as pl
from jax.experimental.pallas import tpu as pltpu
from jax.experimental.pallas import tpu_sc as plsc

_NW = 32
_CHUNK = 80
_NB = 5


def _sc_mesh():
    return plsc.VectorSubcoreMesh(core_axis_name="c", subcore_axis_name="s")


def _sc_deg(dst2d, zeros_n):
    nch = dst2d.shape[0]
    n = zeros_n.shape[0]
    cpw = nch // _NW
    nsteps = cpw // _NB

    @functools.partial(
        pl.kernel,
        out_type=jax.ShapeDtypeStruct((2 * n,), jnp.float32),
        mesh=_sc_mesh(),
        compiler_params=pltpu.CompilerParams(use_tc_tiling_on_sc=False),
        scratch_types=[
            pltpu.VMEM((cpw, _CHUNK), jnp.int32),
            pltpu.VMEM((_CHUNK,), jnp.float32),
            pltpu.VMEM((n,), jnp.float32),
            pltpu.VMEM_SHARED((n,), jnp.float32),
        ] + [pltpu.SemaphoreType.DMA] * (_NB + 1),
    )
    def k(dst_hbm, z_hbm, out_hbm, di_all, ones_v, buf_v, acc_sh, *sems):
        ssem = sems[:_NB]
        isem = sems[_NB]
        cid = lax.axis_index("c")
        sid = lax.axis_index("s")
        wid = sid * 2 + cid
        c0 = wid * cpw
        ip = pltpu.async_copy(dst_hbm.at[pl.ds(c0, cpw)], di_all, isem)
        for i in range(_CHUNK // 16):
            ones_v[pl.ds(i * 16, 16)] = jnp.ones((16,), jnp.float32)

        @pl.when(sid == 0)
        def _():
            pltpu.sync_copy(z_hbm, buf_v)
            pltpu.sync_copy(buf_v, acc_sh)

        ip.wait()
        plsc.subcore_barrier()

        def step(j, carry):
            @pl.when(j > 0)
            def _():
                for b in range(_NB):
                    pltpu.make_async_copy(ones_v, acc_sh.at[di_all.at[0]],
                                          ssem[b]).wait()
            for b in range(_NB):
                pltpu.async_copy(
                    ones_v, acc_sh.at[di_all.at[j * _NB + b]], ssem[b],
                    add=True)
            return carry

        lax.fori_loop(0, nsteps, step, 0)
        for b in range(_NB):
            pltpu.make_async_copy(ones_v, acc_sh.at[di_all.at[0]],
                                  ssem[b]).wait()
        plsc.subcore_barrier()

        @pl.when(sid == 0)
        def _():
            pltpu.sync_copy(acc_sh, buf_v)
            pltpu.sync_copy(buf_v, out_hbm.at[pl.ds(cid * n, n)])

    return k(dst2d, zeros_n)


def _sc_agg(y, src2d, dst2d, zeros_nd):
    n, d = y.shape
    nch = src2d.shape[0]
    cpw = nch // _NW
    nsteps = cpw // _NB
    rpt = (n // 16) // 8 * 8

    @functools.partial(
        pl.kernel,
        out_type=jax.ShapeDtypeStruct((2 * n, d), jnp.float32),
        mesh=_sc_mesh(),
        compiler_params=pltpu.CompilerParams(use_tc_tiling_on_sc=False),
        scratch_types=[
            pltpu.VMEM((cpw, _CHUNK), jnp.int32),
            pltpu.VMEM((cpw, _CHUNK), jnp.int32),
            pltpu.VMEM((_NB, _CHUNK, d), jnp.float32),
            pltpu.VMEM_SHARED((n, d), jnp.float32),
            pltpu.VMEM_SHARED((n, d), jnp.float32),
        ] + [pltpu.SemaphoreType.DMA] * (2 * _NB + 2),
    )
    def k(y_hbm, src_hbm, dst_hbm, z_hbm, out_hbm,
          si_all, di_all, rows, acc_sh, tab_sh, *sems):
        gsem = sems[:_NB]
        ssem = sems[_NB:2 * _NB]
        isem = sems[2 * _NB:]
        cid = lax.axis_index("c")
        sid = lax.axis_index("s")
        wid = sid * 2 + cid
        c0 = wid * cpw
        ca = pltpu.async_copy(src_hbm.at[pl.ds(c0, cpw)], si_all, isem[0])
        cb = pltpu.async_copy(dst_hbm.at[pl.ds(c0, cpw)], di_all, isem[1])

        @pl.when(sid < 15)
        def _():
            r0 = pl.multiple_of(sid * rpt, 8)
            pltpu.sync_copy(y_hbm.at[pl.ds(r0, rpt)],
                            tab_sh.at[pl.ds(r0, rpt)])
            pltpu.sync_copy(z_hbm.at[pl.ds(r0, rpt)],
                            acc_sh.at[pl.ds(r0, rpt)])

        @pl.when(sid == 15)
        def _():
            tail = n - 15 * rpt
            r0 = pl.multiple_of(15 * rpt, 8)
            pltpu.sync_copy(y_hbm.at[pl.ds(r0, tail)],
                            tab_sh.at[pl.ds(r0, tail)])
            pltpu.sync_copy(z_hbm.at[pl.ds(r0, tail)],
                            acc_sh.at[pl.ds(r0, tail)])

        ca.wait()
        cb.wait()
        plsc.subcore_barrier()

        def step(j, carry):
            @pl.when(j > 0)
            def _():
                for b in range(_NB):
                    pltpu.make_async_copy(rows.at[b],
                                          acc_sh.at[di_all.at[0]],
                                          ssem[b]).wait()
            gds = []
            for b in range(_NB):
                i = j * _NB + b
                gds.append(pltpu.async_copy(tab_sh.at[si_all.at[i]],
                                            rows.at[b], gsem[b]))
            for b in range(_NB):
                i = j * _NB + b
                gds[b].wait()
                pltpu.async_copy(rows.at[b], acc_sh.at[di_all.at[i]],
                                 ssem[b], add=True)
            return carry

        lax.fori_loop(0, nsteps, step, 0)
        for b in range(_NB):
            pltpu.make_async_copy(rows.at[b], acc_sh.at[di_all.at[0]],
                                  ssem[b]).wait()
        plsc.subcore_barrier()
        @pl.when(sid < 15)
        def _():
            r0 = pl.multiple_of(sid * rpt, 8)
            o0 = pl.multiple_of(cid * n + sid * rpt, 8)
            pltpu.sync_copy(acc_sh.at[pl.ds(r0, rpt)],
                            out_hbm.at[pl.ds(o0, rpt)])

        @pl.when(sid == 15)
        def _():
            tail = n - 15 * rpt
            o0 = pl.multiple_of(cid * n + 15 * rpt, 8)
            pltpu.sync_copy(acc_sh.at[pl.ds(15 * rpt, tail)],
                            out_hbm.at[pl.ds(o0, tail)])

    return k(y, src2d, dst2d, zeros_nd)


_NBE = 2


def _sc_edge_add(ta, tb, ia2d, ib2d, ef0bc, ef1bc, pk):
    n, d = ta.shape
    npk = pk.shape[0]
    nch = ia2d.shape[0]
    rp = nch * _CHUNK
    cpw = nch // _NW
    nsteps = cpw // _NBE
    rpt = (n // 16) // 8 * 8
    epr = 128 // d
    orows = _CHUNK // epr

    @functools.partial(
        pl.kernel,
        out_type=jax.ShapeDtypeStruct((rp // epr, 128), jnp.float32),
        mesh=_sc_mesh(),
        compiler_params=pltpu.CompilerParams(use_tc_tiling_on_sc=False),
        scratch_types=[
            pltpu.VMEM((cpw, _CHUNK), jnp.int32),
            pltpu.VMEM((cpw, _CHUNK), jnp.int32),
            pltpu.VMEM((_NBE, _CHUNK, d), jnp.float32),
            pltpu.VMEM((_NBE, _CHUNK, d), jnp.float32),
            pltpu.VMEM((_NBE, _CHUNK, 16), jnp.float32),
            pltpu.VMEM((_NBE, _CHUNK, 16), jnp.float32),
            pltpu.VMEM((npk, 16), jnp.float32),
            pltpu.VMEM((_NBE, orows, 128), jnp.float32),
            pltpu.VMEM_SHARED((n, d), jnp.float32),
            pltpu.VMEM_SHARED((n, d), jnp.float32),
        ] + [pltpu.SemaphoreType.DMA] * (5 * _NBE + 2),
    )
    def k(ta_hbm, tb_hbm, ia_hbm, ib_hbm, e0_hbm, e1_hbm, pk_hbm, out_hbm,
          ia_all, ib_all, ra, rb, eb0, eb1, pk_v, ow, tas_sh, tbs_sh, *sems):
        gsa = sems[:_NBE]
        gsb = sems[_NBE:2 * _NBE]
        gse0 = sems[2 * _NBE:3 * _NBE]
        gse1 = sems[3 * _NBE:4 * _NBE]
        ws = sems[4 * _NBE:5 * _NBE]
        isem = sems[5 * _NBE:]
        cid = lax.axis_index("c")
        sid = lax.axis_index("s")
        wid = sid * 2 + cid
        c0 = wid * cpw
        ca = pltpu.async_copy(ia_hbm.at[pl.ds(c0, cpw)], ia_all, isem[0])
        cb = pltpu.async_copy(ib_hbm.at[pl.ds(c0, cpw)], ib_all, isem[1])
        pltpu.sync_copy(pk_hbm, pk_v)

        @pl.when(sid < 15)
        def _():
            r0 = pl.multiple_of(sid * rpt, 8)
            pltpu.sync_copy(ta_hbm.at[pl.ds(r0, rpt)],
                            tas_sh.at[pl.ds(r0, rpt)])
            pltpu.sync_copy(tb_hbm.at[pl.ds(r0, rpt)],
                            tbs_sh.at[pl.ds(r0, rpt)])

        @pl.when(sid == 15)
        def _():
            tail = n - 15 * rpt
            r0 = pl.multiple_of(15 * rpt, 8)
            pltpu.sync_copy(ta_hbm.at[pl.ds(r0, tail)],
                            tas_sh.at[pl.ds(r0, tail)])
            pltpu.sync_copy(tb_hbm.at[pl.ds(r0, tail)],
                            tbs_sh.at[pl.ds(r0, tail)])

        ca.wait()
        cb.wait()
        plsc.subcore_barrier()
        nv = d // 16

        def compute(b):
            wc0 = [pk_v[h, :] for h in range(nv)]
            wc1 = [pk_v[nv + h, :] for h in range(nv)]
            b1v = [pk_v[2 * nv + h, :] for h in range(nv)]
            for e in range(_CHUNK):
                orow = e // epr
                ocol = (e % epr) * d
                e0v = eb0[b, e, pl.ds(0, 16)]
                e1v = eb1[b, e, pl.ds(0, 16)]
                for h in range(nv):
                    ow[b, orow, pl.ds(ocol + h * 16, 16)] = (
                        ra[b, e, pl.ds(h * 16, 16)]
                        + rb[b, e, pl.ds(h * 16, 16)]
                        + e0v * wc0[h] + e1v * wc1[h] + b1v[h])

        def step(j, carry):
            @pl.when(j > 0)
            def _():
                for b in range(_NBE):
                    pltpu.make_async_copy(ow.at[b],
                                          out_hbm.at[pl.ds(0, orows)],
                                          ws[b]).wait()
            gda, gdb, gde0, gde1 = [], [], [], []
            for b in range(_NBE):
                i = j * _NBE + b
                gda.append(pltpu.async_copy(tas_sh.at[ia_all.at[i]],
                                            ra.at[b], gsa[b]))
                gdb.append(pltpu.async_copy(tbs_sh.at[ib_all.at[i]],
                                            rb.at[b], gsb[b]))
                gde0.append(pltpu.async_copy(
                    e0_hbm.at[pl.ds((c0 + i) * _CHUNK, _CHUNK)],
                    eb0.at[b], gse0[b]))
                gde1.append(pltpu.async_copy(
                    e1_hbm.at[pl.ds((c0 + i) * _CHUNK, _CHUNK)],
                    eb1.at[b], gse1[b]))
            for b in range(_NBE):
                i = j * _NBE + b
                o0 = pl.multiple_of((c0 + i) * orows, 4)
                gda[b].wait()
                gdb[b].wait()
                gde0[b].wait()
                gde1[b].wait()
                compute(b)
                pltpu.async_copy(ow.at[b], out_hbm.at[pl.ds(o0, orows)],
                                 ws[b])
            return carry

        lax.fori_loop(0, nsteps, step, 0)
        for b in range(_NBE):
            pltpu.make_async_copy(ow.at[b], out_hbm.at[pl.ds(0, orows)],
                                  ws[b]).wait()

    return k(ta, tb, ia2d, ib2d, ef0bc, ef1bc, pk)


def _sc_gather2(ta, tb, ia2d, ib2d):
    n, d = ta.shape
    nch = ia2d.shape[0]
    rp = nch * _CHUNK
    cpw = nch // _NW
    nsteps = cpw // _NB
    rpt = (n // 16) // 8 * 8

    @functools.partial(
        pl.kernel,
        out_type=(jax.ShapeDtypeStruct((rp, d), jnp.float32),
                  jax.ShapeDtypeStruct((rp, d), jnp.float32)),
        mesh=_sc_mesh(),
        compiler_params=pltpu.CompilerParams(use_tc_tiling_on_sc=False),
        scratch_types=[
            pltpu.VMEM((cpw, _CHUNK), jnp.int32),
            pltpu.VMEM((cpw, _CHUNK), jnp.int32),
            pltpu.VMEM((_NB, _CHUNK, d), jnp.float32),
            pltpu.VMEM((_NB, _CHUNK, d), jnp.float32),
            pltpu.VMEM_SHARED((n, d), jnp.float32),
            pltpu.VMEM_SHARED((n, d), jnp.float32),
        ] + [pltpu.SemaphoreType.DMA] * (4 * _NB + 2),
    )
    def k(ta_hbm, tb_hbm, ia_hbm, ib_hbm, oa_hbm, ob_hbm,
          ia_all, ib_all, ra, rb, tas_sh, tbs_sh, *sems):
        gsa = sems[:_NB]
        gsb = sems[_NB:2 * _NB]
        wsa = sems[2 * _NB:3 * _NB]
        wsb = sems[3 * _NB:4 * _NB]
        isem = sems[4 * _NB:]
        cid = lax.axis_index("c")
        sid = lax.axis_index("s")
        wid = sid * 2 + cid
        c0 = wid * cpw
        ca = pltpu.async_copy(ia_hbm.at[pl.ds(c0, cpw)], ia_all, isem[0])
        cb = pltpu.async_copy(ib_hbm.at[pl.ds(c0, cpw)], ib_all, isem[1])

        @pl.when(sid < 15)
        def _():
            r0 = pl.multiple_of(sid * rpt, 8)
            pltpu.sync_copy(ta_hbm.at[pl.ds(r0, rpt)],
                            tas_sh.at[pl.ds(r0, rpt)])
            pltpu.sync_copy(tb_hbm.at[pl.ds(r0, rpt)],
                            tbs_sh.at[pl.ds(r0, rpt)])

        @pl.when(sid == 15)
        def _():
            tail = n - 15 * rpt
            r0 = pl.multiple_of(15 * rpt, 8)
            pltpu.sync_copy(ta_hbm.at[pl.ds(r0, tail)],
                            tas_sh.at[pl.ds(r0, tail)])
            pltpu.sync_copy(tb_hbm.at[pl.ds(r0, tail)],
                            tbs_sh.at[pl.ds(r0, tail)])

        ca.wait()
        cb.wait()
        plsc.subcore_barrier()

        def step(j, carry):
            @pl.when(j > 0)
            def _():
                for b in range(_NB):
                    pltpu.make_async_copy(ra.at[b],
                                          oa_hbm.at[pl.ds(0, _CHUNK)],
                                          wsa[b]).wait()
                    pltpu.make_async_copy(rb.at[b],
                                          ob_hbm.at[pl.ds(0, _CHUNK)],
                                          wsb[b]).wait()
            gda, gdb = [], []
            for b in range(_NB):
                i = j * _NB + b
                gda.append(pltpu.async_copy(tas_sh.at[ia_all.at[i]],
                                            ra.at[b], gsa[b]))
                gdb.append(pltpu.async_copy(tbs_sh.at[ib_all.at[i]],
                                            rb.at[b], gsb[b]))
            for b in range(_NB):
                i = j * _NB + b
                o0 = pl.multiple_of((c0 + i) * _CHUNK, 8)
                gda[b].wait()
                gdb[b].wait()
                pltpu.async_copy(ra.at[b], oa_hbm.at[pl.ds(o0, _CHUNK)],
                                 wsa[b])
                pltpu.async_copy(rb.at[b], ob_hbm.at[pl.ds(o0, _CHUNK)],
                                 wsb[b])
            return carry

        lax.fori_loop(0, nsteps, step, 0)
        for b in range(_NB):
            pltpu.make_async_copy(ra.at[b], oa_hbm.at[pl.ds(0, _CHUNK)],
                                  wsa[b]).wait()
            pltpu.make_async_copy(rb.at[b], ob_hbm.at[pl.ds(0, _CHUNK)],
                                  wsb[b]).wait()

    return k(ta, tb, ia2d, ib2d, ef0bc, ef1bc, pk)


_BN = 1000
_BR = 2048


def _tc1_body(dp_ref, x_ref, w_ref, o_ref):
    deg = dp_ref[:, 0] + dp_ref[:, 1] + 1.0
    dinv = lax.rsqrt(deg)
    xw = jnp.dot(x_ref[...], w_ref[...], preferred_element_type=jnp.float32)
    o_ref[...] = xw * dinv[:, None]


def _tc1(dp, x, w1):
    n, di = x.shape
    dh = w1.shape[1]
    return pl.pallas_call(
        _tc1_body,
        grid=(n // _BN,),
        in_specs=[pl.BlockSpec((_BN, 2), lambda i: (i, 0)),
                  pl.BlockSpec((_BN, di), lambda i: (i, 0)),
                  pl.BlockSpec((di, dh), lambda i: (0, 0))],
        out_specs=pl.BlockSpec((_BN, dh), lambda i: (i, 0)),
        out_shape=jax.ShapeDtypeStruct((n, dh), jnp.float32),
    )(dp, x, w1)


def _tc2_body(dp_ref, y_ref, pa_ref, pb_ref, b_ref, w_ref, o_ref):
    deg = dp_ref[:, 0] + dp_ref[:, 1] + 1.0
    dinv = lax.rsqrt(deg)
    h = jnp.maximum(
        dinv[:, None] * (pa_ref[...] + pb_ref[...] + y_ref[...]) + b_ref[...],
        0.0)
    o_ref[...] = jnp.dot(h, w_ref[...],
                         preferred_element_type=jnp.float32) * dinv[:, None]


def _tc2(dp, y, pa, pb, b, w2):
    n, dh = y.shape
    return pl.pallas_call(
        _tc2_body,
        grid=(n // _BN,),
        in_specs=[pl.BlockSpec((_BN, 2), lambda i: (i, 0)),
                  pl.BlockSpec((_BN, dh), lambda i: (i, 0)),
                  pl.BlockSpec((_BN, dh), lambda i: (i, 0)),
                  pl.BlockSpec((_BN, dh), lambda i: (i, 0)),
                  pl.BlockSpec((1, dh), lambda i: (0, 0)),
                  pl.BlockSpec((dh, dh), lambda i: (0, 0))],
        out_specs=pl.BlockSpec((_BN, dh), lambda i: (i, 0)),
        out_shape=jax.ShapeDtypeStruct((n, dh), jnp.float32),
    )(dp, y, pa, pb, b, w2)


def _tc3_body(dp_ref, y_ref, pa_ref, pb_ref, b_ref, wa_ref, wb_ref,
              oa_ref, ob_ref):
    deg = dp_ref[:, 0] + dp_ref[:, 1] + 1.0
    dinv = lax.rsqrt(deg)
    h = jnp.maximum(
        dinv[:, None] * (pa_ref[...] + pb_ref[...] + y_ref[...]) + b_ref[...],
        0.0)
    oa_ref[...] = jnp.dot(h, wa_ref[...], preferred_element_type=jnp.float32)
    ob_ref[...] = jnp.dot(h, wb_ref[...], preferred_element_type=jnp.float32)


def _tc3(dp, y, pa, pb, b, wa, wb):
    n, dh = y.shape
    return pl.pallas_call(
        _tc3_body,
        grid=(n // _BN,),
        in_specs=[pl.BlockSpec((_BN, 2), lambda i: (i, 0)),
                  pl.BlockSpec((_BN, dh), lambda i: (i, 0)),
                  pl.BlockSpec((_BN, dh), lambda i: (i, 0)),
                  pl.BlockSpec((_BN, dh), lambda i: (i, 0)),
                  pl.BlockSpec((1, dh), lambda i: (0, 0)),
                  pl.BlockSpec((dh, dh), lambda i: (0, 0)),
                  pl.BlockSpec((dh, dh), lambda i: (0, 0))],
        out_specs=[pl.BlockSpec((_BN, dh), lambda i: (i, 0)),
                   pl.BlockSpec((_BN, dh), lambda i: (i, 0))],
        out_shape=[jax.ShapeDtypeStruct((n, dh), jnp.float32),
                   jax.ShapeDtypeStruct((n, dh), jnp.float32)],
    )(dp, y, pa, pb, b, wa, wb)


def _tc4_body(sp_ref, w2_ref, b2_ref, o_ref):
    x = jnp.reshape(sp_ref[...], (_BR, 128))
    cols = []
    for k in range(4):
        h = jnp.maximum(x[:, k * 32:(k + 1) * 32], 0.0)
        cols.append(jnp.dot(h, w2_ref[...],
                            preferred_element_type=jnp.float32))
    o_ref[...] = jnp.concatenate(cols, axis=1) + b2_ref[...]


def _tc4(sp_flat, w2, b2):
    rq = sp_flat.shape[0] // 128
    dh = w2.shape[0]
    return pl.pallas_call(
        _tc4_body,
        grid=(rq // _BR,),
        in_specs=[pl.BlockSpec((_BR * 128,), lambda i: (i,)),
                  pl.BlockSpec((dh, 1), lambda i: (0, 0)),
                  pl.BlockSpec((1, 1), lambda i: (0, 0))],
        out_specs=pl.BlockSpec((_BR, 4), lambda i: (i, 0)),
        out_shape=jax.ShapeDtypeStruct((rq, 4), jnp.float32),
    )(sp_flat, w2, b2)


def kernel(x, edge_index, tf_edge_idx, gene_edge_idx, edge_features,
           W1, b1, W2, b2, We1, be1, We2, be2):
    n, _ = x.shape
    dh = W1.shape[1]
    ei = edge_index.astype(jnp.int32)
    src2d = ei[0].reshape(-1, _CHUNK)
    dst2d = ei[1].reshape(-1, _CHUNK)
    tf_i = tf_edge_idx.astype(jnp.int32)
    ge_i = gene_edge_idx.astype(jnp.int32)
    r = tf_i.shape[0]

    gran = _NW * _CHUNK * _NBE
    rp = ((r + gran - 1) // gran) * gran
    pad = rp - r
    tf2d = jnp.pad(tf_i, (0, pad)).reshape(-1, _CHUNK)
    ge2d = jnp.pad(ge_i, (0, pad)).reshape(-1, _CHUNK)
    ef_p = jnp.pad(edge_features, ((0, pad), (0, 0)))
    ef_rt = ef_p.astype(jnp.bfloat16).astype(jnp.float32)
    ef0bc = jnp.repeat(ef_rt[:, 0:1], 16, 1)
    ef1bc = jnp.repeat(ef_rt[:, 1:2], 16, 1)

    z1 = jnp.zeros((n,), jnp.float32)
    z2 = jnp.zeros((n, dh), jnp.float32)

    degp = _sc_deg(dst2d, z1).reshape(2, n).T
    y1 = _tc1(degp, x, W1)
    p1 = _sc_agg(y1, src2d, dst2d, z2)
    y2 = _tc2(degp, y1, p1[:n], p1[n:], b1.reshape(1, dh), W2)
    p2 = _sc_agg(y2, src2d, dst2d, z2)
    gtf, gge = _tc3(degp, y2, p2[:n], p2[n:], b2.reshape(1, dh),
                    We1[:dh], We1[dh:2 * dh])
    wc = We1[2 * dh:]
    wc_rt = wc.astype(jnp.bfloat16).astype(jnp.float32)
    pk = jnp.concatenate([wc_rt[0].reshape(2, 16), wc_rt[1].reshape(2, 16),
                          be1.reshape(2, 16)], axis=0)
    spk = _sc_edge_add(gtf, gge, tf2d, ge2d, ef0bc, ef1bc, pk)
    pred4 = _tc4(spk.reshape(-1), We2, be2.reshape(1, 1))
    return pred4.reshape(-1)[:r]


# speedup vs baseline: 1.2035x; 1.2035x over previous
import functools
import math

import jax
import jax.numpy as jnp
from jax import lax
from jax.experimental import pallas as pl
from jax.experimental.pallas import tpu as pltpu
from jax.experimental.pallas import tpu_sc as plsc

_NW = 32
_CHUNK = 80
_NB = 5


def _sc_mesh():
    return plsc.VectorSubcoreMesh(core_axis_name="c", subcore_axis_name="s")


def _sc_deg(dst2d, zeros_n):
    nch = dst2d.shape[0]
    n = zeros_n.shape[0]
    cpw = nch // _NW
    nsteps = cpw // _NB

    @functools.partial(
        pl.kernel,
        out_type=jax.ShapeDtypeStruct((2 * n,), jnp.float32),
        mesh=_sc_mesh(),
        compiler_params=pltpu.CompilerParams(use_tc_tiling_on_sc=False),
        scratch_types=[
            pltpu.VMEM((cpw, _CHUNK), jnp.int32),
            pltpu.VMEM((_CHUNK,), jnp.float32),
            pltpu.VMEM((n,), jnp.float32),
            pltpu.VMEM_SHARED((n,), jnp.float32),
        ] + [pltpu.SemaphoreType.DMA] * (_NB + 1),
    )
    def k(dst_hbm, z_hbm, out_hbm, di_all, ones_v, buf_v, acc_sh, *sems):
        ssem = sems[:_NB]
        isem = sems[_NB]
        cid = lax.axis_index("c")
        sid = lax.axis_index("s")
        wid = sid * 2 + cid
        c0 = wid * cpw
        ip = pltpu.async_copy(dst_hbm.at[pl.ds(c0, cpw)], di_all, isem)
        for i in range(_CHUNK // 16):
            ones_v[pl.ds(i * 16, 16)] = jnp.ones((16,), jnp.float32)

        @pl.when(sid == 0)
        def _():
            pltpu.sync_copy(z_hbm, buf_v)
            pltpu.sync_copy(buf_v, acc_sh)

        ip.wait()
        plsc.subcore_barrier()

        def step(j, carry):
            @pl.when(j > 0)
            def _():
                for b in range(_NB):
                    pltpu.make_async_copy(ones_v, acc_sh.at[di_all.at[0]],
                                          ssem[b]).wait()
            for b in range(_NB):
                pltpu.async_copy(
                    ones_v, acc_sh.at[di_all.at[j * _NB + b]], ssem[b],
                    add=True)
            return carry

        lax.fori_loop(0, nsteps, step, 0)
        for b in range(_NB):
            pltpu.make_async_copy(ones_v, acc_sh.at[di_all.at[0]],
                                  ssem[b]).wait()
        plsc.subcore_barrier()

        @pl.when(sid == 0)
        def _():
            pltpu.sync_copy(acc_sh, buf_v)
            pltpu.sync_copy(buf_v, out_hbm.at[pl.ds(cid * n, n)])

    return k(dst2d, zeros_n)


def _sc_agg(y, src2d, dst2d, zeros_nd):
    n, d = y.shape
    nch = src2d.shape[0]
    cpw = nch // _NW
    nsteps = cpw // _NB
    rpt = (n // 16) // 8 * 8

    @functools.partial(
        pl.kernel,
        out_type=jax.ShapeDtypeStruct((2 * n, d), jnp.float32),
        mesh=_sc_mesh(),
        compiler_params=pltpu.CompilerParams(use_tc_tiling_on_sc=False),
        scratch_types=[
            pltpu.VMEM((cpw, _CHUNK), jnp.int32),
            pltpu.VMEM((cpw, _CHUNK), jnp.int32),
            pltpu.VMEM((_NB, _CHUNK, d), jnp.float32),
            pltpu.VMEM_SHARED((n, d), jnp.float32),
            pltpu.VMEM_SHARED((n, d), jnp.float32),
        ] + [pltpu.SemaphoreType.DMA] * (2 * _NB + 2),
    )
    def k(y_hbm, src_hbm, dst_hbm, z_hbm, out_hbm,
          si_all, di_all, rows, acc_sh, tab_sh, *sems):
        gsem = sems[:_NB]
        ssem = sems[_NB:2 * _NB]
        isem = sems[2 * _NB:]
        cid = lax.axis_index("c")
        sid = lax.axis_index("s")
        wid = sid * 2 + cid
        c0 = wid * cpw
        ca = pltpu.async_copy(src_hbm.at[pl.ds(c0, cpw)], si_all, isem[0])
        cb = pltpu.async_copy(dst_hbm.at[pl.ds(c0, cpw)], di_all, isem[1])

        @pl.when(sid < 15)
        def _():
            r0 = pl.multiple_of(sid * rpt, 8)
            pltpu.sync_copy(y_hbm.at[pl.ds(r0, rpt)],
                            tab_sh.at[pl.ds(r0, rpt)])
            pltpu.sync_copy(z_hbm.at[pl.ds(r0, rpt)],
                            acc_sh.at[pl.ds(r0, rpt)])

        @pl.when(sid == 15)
        def _():
            tail = n - 15 * rpt
            r0 = pl.multiple_of(15 * rpt, 8)
            pltpu.sync_copy(y_hbm.at[pl.ds(r0, tail)],
                            tab_sh.at[pl.ds(r0, tail)])
            pltpu.sync_copy(z_hbm.at[pl.ds(r0, tail)],
                            acc_sh.at[pl.ds(r0, tail)])

        ca.wait()
        cb.wait()
        plsc.subcore_barrier()

        def step(j, carry):
            @pl.when(j > 0)
            def _():
                for b in range(_NB):
                    pltpu.make_async_copy(rows.at[b],
                                          acc_sh.at[di_all.at[0]],
                                          ssem[b]).wait()
            gds = []
            for b in range(_NB):
                i = j * _NB + b
                gds.append(pltpu.async_copy(tab_sh.at[si_all.at[i]],
                                            rows.at[b], gsem[b]))
            for b in range(_NB):
                i = j * _NB + b
                gds[b].wait()
                pltpu.async_copy(rows.at[b], acc_sh.at[di_all.at[i]],
                                 ssem[b], add=True)
            return carry

        lax.fori_loop(0, nsteps, step, 0)
        for b in range(_NB):
            pltpu.make_async_copy(rows.at[b], acc_sh.at[di_all.at[0]],
                                  ssem[b]).wait()
        plsc.subcore_barrier()
        @pl.when(sid < 15)
        def _():
            r0 = pl.multiple_of(sid * rpt, 8)
            o0 = pl.multiple_of(cid * n + sid * rpt, 8)
            pltpu.sync_copy(acc_sh.at[pl.ds(r0, rpt)],
                            out_hbm.at[pl.ds(o0, rpt)])

        @pl.when(sid == 15)
        def _():
            tail = n - 15 * rpt
            o0 = pl.multiple_of(cid * n + 15 * rpt, 8)
            pltpu.sync_copy(acc_sh.at[pl.ds(15 * rpt, tail)],
                            out_hbm.at[pl.ds(o0, tail)])

    return k(y, src2d, dst2d, zeros_nd)


_NBE = 2


def _sc_edge_add(ta, tb, ia2d, ib2d):
    n, d = ta.shape
    nch = ia2d.shape[0]
    rp = nch * _CHUNK
    cpw = nch // _NW
    nsteps = cpw // _NBE
    rpt = (n // 16) // 8 * 8
    epr = 128 // d
    orows = _CHUNK // epr

    @functools.partial(
        pl.kernel,
        out_type=jax.ShapeDtypeStruct((rp // epr, 128), jnp.float32),
        mesh=_sc_mesh(),
        compiler_params=pltpu.CompilerParams(use_tc_tiling_on_sc=False),
        scratch_types=[
            pltpu.VMEM((cpw, _CHUNK), jnp.int32),
            pltpu.VMEM((cpw, _CHUNK), jnp.int32),
            pltpu.VMEM((_NBE, _CHUNK, d), jnp.float32),
            pltpu.VMEM((_NBE, _CHUNK, d), jnp.float32),
            pltpu.VMEM((_NBE, orows, 128), jnp.float32),
            pltpu.VMEM_SHARED((n, d), jnp.float32),
            pltpu.VMEM_SHARED((n, d), jnp.float32),
        ] + [pltpu.SemaphoreType.DMA] * (3 * _NBE + 2),
    )
    def k(ta_hbm, tb_hbm, ia_hbm, ib_hbm, out_hbm,
          ia_all, ib_all, ra, rb, ow, tas_sh, tbs_sh, *sems):
        gsa = sems[:_NBE]
        gsb = sems[_NBE:2 * _NBE]
        ws = sems[2 * _NBE:3 * _NBE]
        isem = sems[3 * _NBE:]
        cid = lax.axis_index("c")
        sid = lax.axis_index("s")
        wid = sid * 2 + cid
        c0 = wid * cpw
        ca = pltpu.async_copy(ia_hbm.at[pl.ds(c0, cpw)], ia_all, isem[0])
        cb = pltpu.async_copy(ib_hbm.at[pl.ds(c0, cpw)], ib_all, isem[1])

        @pl.when(sid < 15)
        def _():
            r0 = pl.multiple_of(sid * rpt, 8)
            pltpu.sync_copy(ta_hbm.at[pl.ds(r0, rpt)],
                            tas_sh.at[pl.ds(r0, rpt)])
            pltpu.sync_copy(tb_hbm.at[pl.ds(r0, rpt)],
                            tbs_sh.at[pl.ds(r0, rpt)])

        @pl.when(sid == 15)
        def _():
            tail = n - 15 * rpt
            r0 = pl.multiple_of(15 * rpt, 8)
            pltpu.sync_copy(ta_hbm.at[pl.ds(r0, tail)],
                            tas_sh.at[pl.ds(r0, tail)])
            pltpu.sync_copy(tb_hbm.at[pl.ds(r0, tail)],
                            tbs_sh.at[pl.ds(r0, tail)])

        ca.wait()
        cb.wait()
        plsc.subcore_barrier()
        nv = d // 16

        def compute(b):
            for e in range(_CHUNK):
                orow = e // epr
                ocol = (e % epr) * d
                for h in range(nv):
                    ow[b, orow, pl.ds(ocol + h * 16, 16)] = (
                        ra[b, e, pl.ds(h * 16, 16)]
                        + rb[b, e, pl.ds(h * 16, 16)])

        def step(j, carry):
            @pl.when(j > 0)
            def _():
                for b in range(_NBE):
                    pltpu.make_async_copy(ow.at[b],
                                          out_hbm.at[pl.ds(0, orows)],
                                          ws[b]).wait()
            gda, gdb = [], []
            for b in range(_NBE):
                i = j * _NBE + b
                gda.append(pltpu.async_copy(tas_sh.at[ia_all.at[i]],
                                            ra.at[b], gsa[b]))
                gdb.append(pltpu.async_copy(tbs_sh.at[ib_all.at[i]],
                                            rb.at[b], gsb[b]))
            for b in range(_NBE):
                i = j * _NBE + b
                o0 = pl.multiple_of((c0 + i) * orows, 4)
                gda[b].wait()
                gdb[b].wait()
                compute(b)
                pltpu.async_copy(ow.at[b], out_hbm.at[pl.ds(o0, orows)],
                                 ws[b])
            return carry

        lax.fori_loop(0, nsteps, step, 0)
        for b in range(_NBE):
            pltpu.make_async_copy(ow.at[b], out_hbm.at[pl.ds(0, orows)],
                                  ws[b]).wait()

    return k(ta, tb, ia2d, ib2d)


def _sc_gather2(ta, tb, ia2d, ib2d):
    n, d = ta.shape
    nch = ia2d.shape[0]
    rp = nch * _CHUNK
    cpw = nch // _NW
    nsteps = cpw // _NB
    rpt = (n // 16) // 8 * 8

    @functools.partial(
        pl.kernel,
        out_type=(jax.ShapeDtypeStruct((rp, d), jnp.float32),
                  jax.ShapeDtypeStruct((rp, d), jnp.float32)),
        mesh=_sc_mesh(),
        compiler_params=pltpu.CompilerParams(use_tc_tiling_on_sc=False),
        scratch_types=[
            pltpu.VMEM((cpw, _CHUNK), jnp.int32),
            pltpu.VMEM((cpw, _CHUNK), jnp.int32),
            pltpu.VMEM((_NB, _CHUNK, d), jnp.float32),
            pltpu.VMEM((_NB, _CHUNK, d), jnp.float32),
            pltpu.VMEM_SHARED((n, d), jnp.float32),
            pltpu.VMEM_SHARED((n, d), jnp.float32),
        ] + [pltpu.SemaphoreType.DMA] * (4 * _NB + 2),
    )
    def k(ta_hbm, tb_hbm, ia_hbm, ib_hbm, oa_hbm, ob_hbm,
          ia_all, ib_all, ra, rb, tas_sh, tbs_sh, *sems):
        gsa = sems[:_NB]
        gsb = sems[_NB:2 * _NB]
        wsa = sems[2 * _NB:3 * _NB]
        wsb = sems[3 * _NB:4 * _NB]
        isem = sems[4 * _NB:]
        cid = lax.axis_index("c")
        sid = lax.axis_index("s")
        wid = sid * 2 + cid
        c0 = wid * cpw
        ca = pltpu.async_copy(ia_hbm.at[pl.ds(c0, cpw)], ia_all, isem[0])
        cb = pltpu.async_copy(ib_hbm.at[pl.ds(c0, cpw)], ib_all, isem[1])

        @pl.when(sid < 15)
        def _():
            r0 = pl.multiple_of(sid * rpt, 8)
            pltpu.sync_copy(ta_hbm.at[pl.ds(r0, rpt)],
                            tas_sh.at[pl.ds(r0, rpt)])
            pltpu.sync_copy(tb_hbm.at[pl.ds(r0, rpt)],
                            tbs_sh.at[pl.ds(r0, rpt)])

        @pl.when(sid == 15)
        def _():
            tail = n - 15 * rpt
            r0 = pl.multiple_of(15 * rpt, 8)
            pltpu.sync_copy(ta_hbm.at[pl.ds(r0, tail)],
                            tas_sh.at[pl.ds(r0, tail)])
            pltpu.sync_copy(tb_hbm.at[pl.ds(r0, tail)],
                            tbs_sh.at[pl.ds(r0, tail)])

        ca.wait()
        cb.wait()
        plsc.subcore_barrier()

        def step(j, carry):
            @pl.when(j > 0)
            def _():
                for b in range(_NB):
                    pltpu.make_async_copy(ra.at[b],
                                          oa_hbm.at[pl.ds(0, _CHUNK)],
                                          wsa[b]).wait()
                    pltpu.make_async_copy(rb.at[b],
                                          ob_hbm.at[pl.ds(0, _CHUNK)],
                                          wsb[b]).wait()
            gda, gdb = [], []
            for b in range(_NB):
                i = j * _NB + b
                gda.append(pltpu.async_copy(tas_sh.at[ia_all.at[i]],
                                            ra.at[b], gsa[b]))
                gdb.append(pltpu.async_copy(tbs_sh.at[ib_all.at[i]],
                                            rb.at[b], gsb[b]))
            for b in range(_NB):
                i = j * _NB + b
                o0 = pl.multiple_of((c0 + i) * _CHUNK, 8)
                gda[b].wait()
                gdb[b].wait()
                pltpu.async_copy(ra.at[b], oa_hbm.at[pl.ds(o0, _CHUNK)],
                                 wsa[b])
                pltpu.async_copy(rb.at[b], ob_hbm.at[pl.ds(o0, _CHUNK)],
                                 wsb[b])
            return carry

        lax.fori_loop(0, nsteps, step, 0)
        for b in range(_NB):
            pltpu.make_async_copy(ra.at[b], oa_hbm.at[pl.ds(0, _CHUNK)],
                                  wsa[b]).wait()
            pltpu.make_async_copy(rb.at[b], ob_hbm.at[pl.ds(0, _CHUNK)],
                                  wsb[b]).wait()

    return k(ta, tb, ia2d, ib2d)


_BN = 1000
_BR = 2048


def _tc1_body(dp_ref, x_ref, w_ref, o_ref):
    deg = dp_ref[:, 0] + dp_ref[:, 1] + 1.0
    dinv = lax.rsqrt(deg)
    xw = jnp.dot(x_ref[...], w_ref[...], preferred_element_type=jnp.float32)
    o_ref[...] = xw * dinv[:, None]


def _tc1(dp, x, w1):
    n, di = x.shape
    dh = w1.shape[1]
    return pl.pallas_call(
        _tc1_body,
        grid=(n // _BN,),
        in_specs=[pl.BlockSpec((_BN, 2), lambda i: (i, 0)),
                  pl.BlockSpec((_BN, di), lambda i: (i, 0)),
                  pl.BlockSpec((di, dh), lambda i: (0, 0))],
        out_specs=pl.BlockSpec((_BN, dh), lambda i: (i, 0)),
        out_shape=jax.ShapeDtypeStruct((n, dh), jnp.float32),
    )(dp, x, w1)


def _tc2_body(dp_ref, y_ref, pa_ref, pb_ref, b_ref, w_ref, o_ref):
    deg = dp_ref[:, 0] + dp_ref[:, 1] + 1.0
    dinv = lax.rsqrt(deg)
    h = jnp.maximum(
        dinv[:, None] * (pa_ref[...] + pb_ref[...] + y_ref[...]) + b_ref[...],
        0.0)
    o_ref[...] = jnp.dot(h, w_ref[...],
                         preferred_element_type=jnp.float32) * dinv[:, None]


def _tc2(dp, y, pa, pb, b, w2):
    n, dh = y.shape
    return pl.pallas_call(
        _tc2_body,
        grid=(n // _BN,),
        in_specs=[pl.BlockSpec((_BN, 2), lambda i: (i, 0)),
                  pl.BlockSpec((_BN, dh), lambda i: (i, 0)),
                  pl.BlockSpec((_BN, dh), lambda i: (i, 0)),
                  pl.BlockSpec((_BN, dh), lambda i: (i, 0)),
                  pl.BlockSpec((1, dh), lambda i: (0, 0)),
                  pl.BlockSpec((dh, dh), lambda i: (0, 0))],
        out_specs=pl.BlockSpec((_BN, dh), lambda i: (i, 0)),
        out_shape=jax.ShapeDtypeStruct((n, dh), jnp.float32),
    )(dp, y, pa, pb, b, w2)


def _tc3_body(dp_ref, y_ref, pa_ref, pb_ref, b_ref, wa_ref, wb_ref,
              oa_ref, ob_ref):
    deg = dp_ref[:, 0] + dp_ref[:, 1] + 1.0
    dinv = lax.rsqrt(deg)
    h = jnp.maximum(
        dinv[:, None] * (pa_ref[...] + pb_ref[...] + y_ref[...]) + b_ref[...],
        0.0)
    oa_ref[...] = jnp.dot(h, wa_ref[...], preferred_element_type=jnp.float32)
    ob_ref[...] = jnp.dot(h, wb_ref[...], preferred_element_type=jnp.float32)


def _tc3(dp, y, pa, pb, b, wa, wb):
    n, dh = y.shape
    return pl.pallas_call(
        _tc3_body,
        grid=(n // _BN,),
        in_specs=[pl.BlockSpec((_BN, 2), lambda i: (i, 0)),
                  pl.BlockSpec((_BN, dh), lambda i: (i, 0)),
                  pl.BlockSpec((_BN, dh), lambda i: (i, 0)),
                  pl.BlockSpec((_BN, dh), lambda i: (i, 0)),
                  pl.BlockSpec((1, dh), lambda i: (0, 0)),
                  pl.BlockSpec((dh, dh), lambda i: (0, 0)),
                  pl.BlockSpec((dh, dh), lambda i: (0, 0))],
        out_specs=[pl.BlockSpec((_BN, dh), lambda i: (i, 0)),
                   pl.BlockSpec((_BN, dh), lambda i: (i, 0))],
        out_shape=[jax.ShapeDtypeStruct((n, dh), jnp.float32),
                   jax.ShapeDtypeStruct((n, dh), jnp.float32)],
    )(dp, y, pa, pb, b, wa, wb)


def _tc4_body(sp_ref, ef_ref, wc_ref, b1_ref, w2_ref, b2_ref, o_ref):
    x = jnp.reshape(sp_ref[...], (_BR, 128))
    ef = ef_ref[...]
    wc = wc_ref[...]
    cols = []
    for k in range(4):
        c = (x[:, k * 32:(k + 1) * 32]
             + jnp.dot(ef[:, 2 * k:2 * k + 2], wc,
                       preferred_element_type=jnp.float32)
             + b1_ref[...])
        h = jnp.maximum(c, 0.0)
        cols.append(jnp.dot(h, w2_ref[...],
                            preferred_element_type=jnp.float32))
    o_ref[...] = jnp.concatenate(cols, axis=1) + b2_ref[...]


def _tc4(sp_flat, efp, wc, b1, w2, b2):
    rq = sp_flat.shape[0] // 128
    dh = wc.shape[1]
    return pl.pallas_call(
        _tc4_body,
        grid=(rq // _BR,),
        in_specs=[pl.BlockSpec((_BR * 128,), lambda i: (i,)),
                  pl.BlockSpec((_BR, 8), lambda i: (i, 0)),
                  pl.BlockSpec((2, dh), lambda i: (0, 0)),
                  pl.BlockSpec((1, dh), lambda i: (0, 0)),
                  pl.BlockSpec((dh, 1), lambda i: (0, 0)),
                  pl.BlockSpec((1, 1), lambda i: (0, 0))],
        out_specs=pl.BlockSpec((_BR, 4), lambda i: (i, 0)),
        out_shape=jax.ShapeDtypeStruct((rq, 4), jnp.float32),
    )(sp_flat, efp, wc, b1, w2, b2)


def kernel(x, edge_index, tf_edge_idx, gene_edge_idx, edge_features,
           W1, b1, W2, b2, We1, be1, We2, be2):
    n, _ = x.shape
    dh = W1.shape[1]
    ei = edge_index.astype(jnp.int32)
    src2d = ei[0].reshape(-1, _CHUNK)
    dst2d = ei[1].reshape(-1, _CHUNK)
    tf_i = tf_edge_idx.astype(jnp.int32)
    ge_i = gene_edge_idx.astype(jnp.int32)
    r = tf_i.shape[0]

    gran = _NW * _CHUNK * _NBE
    rp = ((r + gran - 1) // gran) * gran
    pad = rp - r
    tf2d = jnp.pad(tf_i, (0, pad)).reshape(-1, _CHUNK)
    ge2d = jnp.pad(ge_i, (0, pad)).reshape(-1, _CHUNK)
    ef_pk = jnp.pad(edge_features, ((0, pad), (0, 0))).reshape(-1, 8)

    z1 = jnp.zeros((n,), jnp.float32)
    z2 = jnp.zeros((n, dh), jnp.float32)

    degp = _sc_deg(dst2d, z1).reshape(2, n).T
    y1 = _tc1(degp, x, W1)
    p1 = _sc_agg(y1, src2d, dst2d, z2)
    y2 = _tc2(degp, y1, p1[:n], p1[n:], b1.reshape(1, dh), W2)
    p2 = _sc_agg(y2, src2d, dst2d, z2)
    gtf, gge = _tc3(degp, y2, p2[:n], p2[n:], b2.reshape(1, dh),
                    We1[:dh], We1[dh:2 * dh])
    spk = _sc_edge_add(gtf, gge, tf2d, ge2d)
    pred4 = _tc4(spk.reshape(-1), ef_pk, We1[2 * dh:], be1.reshape(1, dh),
                 We2, be2.reshape(1, 1))
    return pred4.reshape(-1)[:r]


# speedup vs baseline: 1.2039x; 1.0003x over previous
import functools
import math

import jax
import jax.numpy as jnp
from jax import lax
from jax.experimental import pallas as pl
from jax.experimental.pallas import tpu as pltpu
from jax.experimental.pallas import tpu_sc as plsc

_NW = 32
_CHUNK = 80
_NB = 5


def _sc_mesh():
    return plsc.VectorSubcoreMesh(core_axis_name="c", subcore_axis_name="s")


def _sc_deg(dst2d, zeros_n):
    nch = dst2d.shape[0]
    n = zeros_n.shape[0]
    cpw = nch // _NW
    nsteps = cpw // _NB

    @functools.partial(
        pl.kernel,
        out_type=jax.ShapeDtypeStruct((2 * n,), jnp.float32),
        mesh=_sc_mesh(),
        compiler_params=pltpu.CompilerParams(use_tc_tiling_on_sc=False),
        scratch_types=[
            pltpu.VMEM((cpw, _CHUNK), jnp.int32),
            pltpu.VMEM((_CHUNK,), jnp.float32),
            pltpu.VMEM((n,), jnp.float32),
            pltpu.VMEM_SHARED((n,), jnp.float32),
        ] + [pltpu.SemaphoreType.DMA] * (_NB + 1),
    )
    def k(dst_hbm, z_hbm, out_hbm, di_all, ones_v, buf_v, acc_sh, *sems):
        ssem = sems[:_NB]
        isem = sems[_NB]
        cid = lax.axis_index("c")
        sid = lax.axis_index("s")
        wid = sid * 2 + cid
        c0 = wid * cpw
        ip = pltpu.async_copy(dst_hbm.at[pl.ds(c0, cpw)], di_all, isem)
        for i in range(_CHUNK // 16):
            ones_v[pl.ds(i * 16, 16)] = jnp.ones((16,), jnp.float32)

        @pl.when(sid == 0)
        def _():
            pltpu.sync_copy(z_hbm, buf_v)
            pltpu.sync_copy(buf_v, acc_sh)

        ip.wait()
        plsc.subcore_barrier()

        def step(j, carry):
            @pl.when(j > 0)
            def _():
                for b in range(_NB):
                    pltpu.make_async_copy(ones_v, acc_sh.at[di_all.at[0]],
                                          ssem[b]).wait()
            for b in range(_NB):
                pltpu.async_copy(
                    ones_v, acc_sh.at[di_all.at[j * _NB + b]], ssem[b],
                    add=True)
            return carry

        lax.fori_loop(0, nsteps, step, 0)
        for b in range(_NB):
            pltpu.make_async_copy(ones_v, acc_sh.at[di_all.at[0]],
                                  ssem[b]).wait()
        plsc.subcore_barrier()

        @pl.when(sid == 0)
        def _():
            pltpu.sync_copy(acc_sh, buf_v)
            pltpu.sync_copy(buf_v, out_hbm.at[pl.ds(cid * n, n)])

    return k(dst2d, zeros_n)


def _sc_agg(y, src2d, dst2d, zeros_nd):
    n, d = y.shape
    nch = src2d.shape[0]
    cpw = nch // _NW
    nsteps = cpw // _NB
    rpt = (n // 16) // 8 * 8

    @functools.partial(
        pl.kernel,
        out_type=jax.ShapeDtypeStruct((2 * n, d), jnp.float32),
        mesh=_sc_mesh(),
        compiler_params=pltpu.CompilerParams(use_tc_tiling_on_sc=False),
        scratch_types=[
            pltpu.VMEM((cpw, _CHUNK), jnp.int32),
            pltpu.VMEM((cpw, _CHUNK), jnp.int32),
            pltpu.VMEM((_NB, _CHUNK, d), jnp.float32),
            pltpu.VMEM_SHARED((n, d), jnp.float32),
            pltpu.VMEM_SHARED((n, d), jnp.float32),
        ] + [pltpu.SemaphoreType.DMA] * (2 * _NB + 2),
    )
    def k(y_hbm, src_hbm, dst_hbm, z_hbm, out_hbm,
          si_all, di_all, rows, acc_sh, tab_sh, *sems):
        gsem = sems[:_NB]
        ssem = sems[_NB:2 * _NB]
        isem = sems[2 * _NB:]
        cid = lax.axis_index("c")
        sid = lax.axis_index("s")
        wid = sid * 2 + cid
        c0 = wid * cpw
        ca = pltpu.async_copy(src_hbm.at[pl.ds(c0, cpw)], si_all, isem[0])
        cb = pltpu.async_copy(dst_hbm.at[pl.ds(c0, cpw)], di_all, isem[1])

        @pl.when(sid < 15)
        def _():
            r0 = pl.multiple_of(sid * rpt, 8)
            pltpu.sync_copy(y_hbm.at[pl.ds(r0, rpt)],
                            tab_sh.at[pl.ds(r0, rpt)])
            pltpu.sync_copy(z_hbm.at[pl.ds(r0, rpt)],
                            acc_sh.at[pl.ds(r0, rpt)])

        @pl.when(sid == 15)
        def _():
            tail = n - 15 * rpt
            r0 = pl.multiple_of(15 * rpt, 8)
            pltpu.sync_copy(y_hbm.at[pl.ds(r0, tail)],
                            tab_sh.at[pl.ds(r0, tail)])
            pltpu.sync_copy(z_hbm.at[pl.ds(r0, tail)],
                            acc_sh.at[pl.ds(r0, tail)])

        ca.wait()
        cb.wait()
        plsc.subcore_barrier()

        def step(j, carry):
            @pl.when(j > 0)
            def _():
                for b in range(_NB):
                    pltpu.make_async_copy(rows.at[b],
                                          acc_sh.at[di_all.at[0]],
                                          ssem[b]).wait()
            gds = []
            for b in range(_NB):
                i = j * _NB + b
                gds.append(pltpu.async_copy(tab_sh.at[si_all.at[i]],
                                            rows.at[b], gsem[b]))
            for b in range(_NB):
                i = j * _NB + b
                gds[b].wait()
                pltpu.async_copy(rows.at[b], acc_sh.at[di_all.at[i]],
                                 ssem[b], add=True)
            return carry

        lax.fori_loop(0, nsteps, step, 0)
        for b in range(_NB):
            pltpu.make_async_copy(rows.at[b], acc_sh.at[di_all.at[0]],
                                  ssem[b]).wait()
        plsc.subcore_barrier()
        @pl.when(sid < 15)
        def _():
            r0 = pl.multiple_of(sid * rpt, 8)
            o0 = pl.multiple_of(cid * n + sid * rpt, 8)
            pltpu.sync_copy(acc_sh.at[pl.ds(r0, rpt)],
                            out_hbm.at[pl.ds(o0, rpt)])

        @pl.when(sid == 15)
        def _():
            tail = n - 15 * rpt
            o0 = pl.multiple_of(cid * n + 15 * rpt, 8)
            pltpu.sync_copy(acc_sh.at[pl.ds(15 * rpt, tail)],
                            out_hbm.at[pl.ds(o0, tail)])

    return k(y, src2d, dst2d, zeros_nd)


_NBE = 2


def _sc_edge_add(ta, tb, ia2d, ib2d):
    n, d = ta.shape
    nch = ia2d.shape[0]
    rp = nch * _CHUNK
    cpw = nch // _NW
    nsteps = cpw // _NBE
    rpt = (n // 16) // 8 * 8
    epr = 128 // d
    orows = _CHUNK // epr

    @functools.partial(
        pl.kernel,
        out_type=jax.ShapeDtypeStruct((rp // epr, 128), jnp.float32),
        mesh=_sc_mesh(),
        compiler_params=pltpu.CompilerParams(use_tc_tiling_on_sc=False),
        scratch_types=[
            pltpu.VMEM((cpw, _CHUNK), jnp.int32),
            pltpu.VMEM((cpw, _CHUNK), jnp.int32),
            pltpu.VMEM((_NBE, _CHUNK, d), jnp.float32),
            pltpu.VMEM((_NBE, _CHUNK, d), jnp.float32),
            pltpu.VMEM((_NBE, orows, 128), jnp.float32),
            pltpu.VMEM_SHARED((n, d), jnp.float32),
            pltpu.VMEM_SHARED((n, d), jnp.float32),
        ] + [pltpu.SemaphoreType.DMA] * (3 * _NBE + 2),
    )
    def k(ta_hbm, tb_hbm, ia_hbm, ib_hbm, out_hbm,
          ia_all, ib_all, ra, rb, ow, tas_sh, tbs_sh, *sems):
        gsa = sems[:_NBE]
        gsb = sems[_NBE:2 * _NBE]
        ws = sems[2 * _NBE:3 * _NBE]
        isem = sems[3 * _NBE:]
        cid = lax.axis_index("c")
        sid = lax.axis_index("s")
        wid = sid * 2 + cid
        c0 = wid * cpw
        ca = pltpu.async_copy(ia_hbm.at[pl.ds(c0, cpw)], ia_all, isem[0])
        cb = pltpu.async_copy(ib_hbm.at[pl.ds(c0, cpw)], ib_all, isem[1])

        @pl.when(sid < 15)
        def _():
            r0 = pl.multiple_of(sid * rpt, 8)
            pltpu.sync_copy(ta_hbm.at[pl.ds(r0, rpt)],
                            tas_sh.at[pl.ds(r0, rpt)])
            pltpu.sync_copy(tb_hbm.at[pl.ds(r0, rpt)],
                            tbs_sh.at[pl.ds(r0, rpt)])

        @pl.when(sid == 15)
        def _():
            tail = n - 15 * rpt
            r0 = pl.multiple_of(15 * rpt, 8)
            pltpu.sync_copy(ta_hbm.at[pl.ds(r0, tail)],
                            tas_sh.at[pl.ds(r0, tail)])
            pltpu.sync_copy(tb_hbm.at[pl.ds(r0, tail)],
                            tbs_sh.at[pl.ds(r0, tail)])

        ca.wait()
        cb.wait()
        plsc.subcore_barrier()
        nv = d // 16

        def compute(b):
            for e in range(_CHUNK):
                orow = e // epr
                ocol = (e % epr) * d
                for h in range(nv):
                    ow[b, orow, pl.ds(ocol + h * 16, 16)] = (
                        ra[b, e, pl.ds(h * 16, 16)]
                        + rb[b, e, pl.ds(h * 16, 16)])

        def step(j, carry):
            @pl.when(j > 0)
            def _():
                for b in range(_NBE):
                    pltpu.make_async_copy(ow.at[b],
                                          out_hbm.at[pl.ds(0, orows)],
                                          ws[b]).wait()
            gda, gdb = [], []
            for b in range(_NBE):
                i = j * _NBE + b
                gda.append(pltpu.async_copy(tas_sh.at[ia_all.at[i]],
                                            ra.at[b], gsa[b]))
                gdb.append(pltpu.async_copy(tbs_sh.at[ib_all.at[i]],
                                            rb.at[b], gsb[b]))
            for b in range(_NBE):
                i = j * _NBE + b
                o0 = pl.multiple_of((c0 + i) * orows, 4)
                gda[b].wait()
                gdb[b].wait()
                compute(b)
                pltpu.async_copy(ow.at[b], out_hbm.at[pl.ds(o0, orows)],
                                 ws[b])
            return carry

        lax.fori_loop(0, nsteps, step, 0)
        for b in range(_NBE):
            pltpu.make_async_copy(ow.at[b], out_hbm.at[pl.ds(0, orows)],
                                  ws[b]).wait()

    return k(ta, tb, ia2d, ib2d)


def _sc_gather2(ta, tb, ia2d, ib2d):
    n, d = ta.shape
    nch = ia2d.shape[0]
    rp = nch * _CHUNK
    cpw = nch // _NW
    nsteps = cpw // _NB
    rpt = (n // 16) // 8 * 8

    @functools.partial(
        pl.kernel,
        out_type=(jax.ShapeDtypeStruct((rp, d), jnp.float32),
                  jax.ShapeDtypeStruct((rp, d), jnp.float32)),
        mesh=_sc_mesh(),
        compiler_params=pltpu.CompilerParams(use_tc_tiling_on_sc=False),
        scratch_types=[
            pltpu.VMEM((cpw, _CHUNK), jnp.int32),
            pltpu.VMEM((cpw, _CHUNK), jnp.int32),
            pltpu.VMEM((_NB, _CHUNK, d), jnp.float32),
            pltpu.VMEM((_NB, _CHUNK, d), jnp.float32),
            pltpu.VMEM_SHARED((n, d), jnp.float32),
            pltpu.VMEM_SHARED((n, d), jnp.float32),
        ] + [pltpu.SemaphoreType.DMA] * (4 * _NB + 2),
    )
    def k(ta_hbm, tb_hbm, ia_hbm, ib_hbm, oa_hbm, ob_hbm,
          ia_all, ib_all, ra, rb, tas_sh, tbs_sh, *sems):
        gsa = sems[:_NB]
        gsb = sems[_NB:2 * _NB]
        wsa = sems[2 * _NB:3 * _NB]
        wsb = sems[3 * _NB:4 * _NB]
        isem = sems[4 * _NB:]
        cid = lax.axis_index("c")
        sid = lax.axis_index("s")
        wid = sid * 2 + cid
        c0 = wid * cpw
        ca = pltpu.async_copy(ia_hbm.at[pl.ds(c0, cpw)], ia_all, isem[0])
        cb = pltpu.async_copy(ib_hbm.at[pl.ds(c0, cpw)], ib_all, isem[1])

        @pl.when(sid < 15)
        def _():
            r0 = pl.multiple_of(sid * rpt, 8)
            pltpu.sync_copy(ta_hbm.at[pl.ds(r0, rpt)],
                            tas_sh.at[pl.ds(r0, rpt)])
            pltpu.sync_copy(tb_hbm.at[pl.ds(r0, rpt)],
                            tbs_sh.at[pl.ds(r0, rpt)])

        @pl.when(sid == 15)
        def _():
            tail = n - 15 * rpt
            r0 = pl.multiple_of(15 * rpt, 8)
            pltpu.sync_copy(ta_hbm.at[pl.ds(r0, tail)],
                            tas_sh.at[pl.ds(r0, tail)])
            pltpu.sync_copy(tb_hbm.at[pl.ds(r0, tail)],
                            tbs_sh.at[pl.ds(r0, tail)])

        ca.wait()
        cb.wait()
        plsc.subcore_barrier()

        def step(j, carry):
            @pl.when(j > 0)
            def _():
                for b in range(_NB):
                    pltpu.make_async_copy(ra.at[b],
                                          oa_hbm.at[pl.ds(0, _CHUNK)],
                                          wsa[b]).wait()
                    pltpu.make_async_copy(rb.at[b],
                                          ob_hbm.at[pl.ds(0, _CHUNK)],
                                          wsb[b]).wait()
            gda, gdb = [], []
            for b in range(_NB):
                i = j * _NB + b
                gda.append(pltpu.async_copy(tas_sh.at[ia_all.at[i]],
                                            ra.at[b], gsa[b]))
                gdb.append(pltpu.async_copy(tbs_sh.at[ib_all.at[i]],
                                            rb.at[b], gsb[b]))
            for b in range(_NB):
                i = j * _NB + b
                o0 = pl.multiple_of((c0 + i) * _CHUNK, 8)
                gda[b].wait()
                gdb[b].wait()
                pltpu.async_copy(ra.at[b], oa_hbm.at[pl.ds(o0, _CHUNK)],
                                 wsa[b])
                pltpu.async_copy(rb.at[b], ob_hbm.at[pl.ds(o0, _CHUNK)],
                                 wsb[b])
            return carry

        lax.fori_loop(0, nsteps, step, 0)
        for b in range(_NB):
            pltpu.make_async_copy(ra.at[b], oa_hbm.at[pl.ds(0, _CHUNK)],
                                  wsa[b]).wait()
            pltpu.make_async_copy(rb.at[b], ob_hbm.at[pl.ds(0, _CHUNK)],
                                  wsb[b]).wait()

    return k(ta, tb, ia2d, ib2d)


_BN = 1000
_BR = 2048


def _tc1_body(dp_ref, x_ref, w_ref, o_ref):
    deg = dp_ref[:, 0] + dp_ref[:, 1] + 1.0
    dinv = lax.rsqrt(deg)
    xw = jnp.dot(x_ref[...], w_ref[...], preferred_element_type=jnp.float32)
    o_ref[...] = xw * dinv[:, None]


def _tc1(dp, x, w1):
    n, di = x.shape
    dh = w1.shape[1]
    return pl.pallas_call(
        _tc1_body,
        grid=(n // _BN,),
        in_specs=[pl.BlockSpec((_BN, 2), lambda i: (i, 0)),
                  pl.BlockSpec((_BN, di), lambda i: (i, 0)),
                  pl.BlockSpec((di, dh), lambda i: (0, 0))],
        out_specs=pl.BlockSpec((_BN, dh), lambda i: (i, 0)),
        out_shape=jax.ShapeDtypeStruct((n, dh), jnp.float32),
    )(dp, x, w1)


def _tc2_body(dp_ref, y_ref, pa_ref, pb_ref, b_ref, w_ref, o_ref):
    deg = dp_ref[:, 0] + dp_ref[:, 1] + 1.0
    dinv = lax.rsqrt(deg)
    h = jnp.maximum(
        dinv[:, None] * (pa_ref[...] + pb_ref[...] + y_ref[...]) + b_ref[...],
        0.0)
    o_ref[...] = jnp.dot(h, w_ref[...],
                         preferred_element_type=jnp.float32) * dinv[:, None]


def _tc2(dp, y, pa, pb, b, w2):
    n, dh = y.shape
    return pl.pallas_call(
        _tc2_body,
        grid=(n // _BN,),
        in_specs=[pl.BlockSpec((_BN, 2), lambda i: (i, 0)),
                  pl.BlockSpec((_BN, dh), lambda i: (i, 0)),
                  pl.BlockSpec((_BN, dh), lambda i: (i, 0)),
                  pl.BlockSpec((_BN, dh), lambda i: (i, 0)),
                  pl.BlockSpec((1, dh), lambda i: (0, 0)),
                  pl.BlockSpec((dh, dh), lambda i: (0, 0))],
        out_specs=pl.BlockSpec((_BN, dh), lambda i: (i, 0)),
        out_shape=jax.ShapeDtypeStruct((n, dh), jnp.float32),
    )(dp, y, pa, pb, b, w2)


def _tc3_body(dp_ref, y_ref, pa_ref, pb_ref, b_ref, wa_ref, wb_ref,
              oa_ref, ob_ref):
    deg = dp_ref[:, 0] + dp_ref[:, 1] + 1.0
    dinv = lax.rsqrt(deg)
    h = jnp.maximum(
        dinv[:, None] * (pa_ref[...] + pb_ref[...] + y_ref[...]) + b_ref[...],
        0.0)
    oa_ref[...] = jnp.dot(h, wa_ref[...], preferred_element_type=jnp.float32)
    ob_ref[...] = jnp.dot(h, wb_ref[...], preferred_element_type=jnp.float32)


def _tc3(dp, y, pa, pb, b, wa, wb):
    n, dh = y.shape
    return pl.pallas_call(
        _tc3_body,
        grid=(n // _BN,),
        in_specs=[pl.BlockSpec((_BN, 2), lambda i: (i, 0)),
                  pl.BlockSpec((_BN, dh), lambda i: (i, 0)),
                  pl.BlockSpec((_BN, dh), lambda i: (i, 0)),
                  pl.BlockSpec((_BN, dh), lambda i: (i, 0)),
                  pl.BlockSpec((1, dh), lambda i: (0, 0)),
                  pl.BlockSpec((dh, dh), lambda i: (0, 0)),
                  pl.BlockSpec((dh, dh), lambda i: (0, 0))],
        out_specs=[pl.BlockSpec((_BN, dh), lambda i: (i, 0)),
                   pl.BlockSpec((_BN, dh), lambda i: (i, 0))],
        out_shape=[jax.ShapeDtypeStruct((n, dh), jnp.float32),
                   jax.ShapeDtypeStruct((n, dh), jnp.float32)],
    )(dp, y, pa, pb, b, wa, wb)


def _tc4_body(sp_ref, ef_ref, wc_ref, b1_ref, w2_ref, b2_ref, o_ref):
    x = sp_ref[...]
    ef = ef_ref[...]
    wc = wc_ref[...]
    cols = []
    for k in range(4):
        c = (x[:, k * 32:(k + 1) * 32]
             + jnp.dot(ef[:, 2 * k:2 * k + 2], wc,
                       preferred_element_type=jnp.float32)
             + b1_ref[...])
        h = jnp.maximum(c, 0.0)
        cols.append(jnp.dot(h, w2_ref[...],
                            preferred_element_type=jnp.float32))
    o_ref[...] = jnp.concatenate(cols, axis=1) + b2_ref[...]


def _tc4(sp, efp, wc, b1, w2, b2):
    rq = sp.shape[0]
    dh = wc.shape[1]
    return pl.pallas_call(
        _tc4_body,
        grid=(rq // _BR,),
        in_specs=[pl.BlockSpec((_BR, 128), lambda i: (i, 0)),
                  pl.BlockSpec((_BR, 8), lambda i: (i, 0)),
                  pl.BlockSpec((2, dh), lambda i: (0, 0)),
                  pl.BlockSpec((1, dh), lambda i: (0, 0)),
                  pl.BlockSpec((dh, 1), lambda i: (0, 0)),
                  pl.BlockSpec((1, 1), lambda i: (0, 0))],
        out_specs=pl.BlockSpec((_BR, 4), lambda i: (i, 0)),
        out_shape=jax.ShapeDtypeStruct((rq, 4), jnp.float32),
    )(sp, efp, wc, b1, w2, b2)


def kernel(x, edge_index, tf_edge_idx, gene_edge_idx, edge_features,
           W1, b1, W2, b2, We1, be1, We2, be2):
    n, _ = x.shape
    dh = W1.shape[1]
    ei = edge_index.astype(jnp.int32)
    src2d = ei[0].reshape(-1, _CHUNK)
    dst2d = ei[1].reshape(-1, _CHUNK)
    tf_i = tf_edge_idx.astype(jnp.int32)
    ge_i = gene_edge_idx.astype(jnp.int32)
    r = tf_i.shape[0]

    gran = _NW * _CHUNK * _NBE
    rp = ((r + gran - 1) // gran) * gran
    pad = rp - r
    tf2d = jnp.pad(tf_i, (0, pad)).reshape(-1, _CHUNK)
    ge2d = jnp.pad(ge_i, (0, pad)).reshape(-1, _CHUNK)
    ef_pk = jnp.pad(edge_features, ((0, pad), (0, 0))).reshape(-1, 8)

    z1 = jnp.zeros((n,), jnp.float32)
    z2 = jnp.zeros((n, dh), jnp.float32)

    degp = _sc_deg(dst2d, z1).reshape(2, n).T
    y1 = _tc1(degp, x, W1)
    p1 = _sc_agg(y1, src2d, dst2d, z2)
    y2 = _tc2(degp, y1, p1[:n], p1[n:], b1.reshape(1, dh), W2)
    p2 = _sc_agg(y2, src2d, dst2d, z2)
    gtf, gge = _tc3(degp, y2, p2[:n], p2[n:], b2.reshape(1, dh),
                    We1[:dh], We1[dh:2 * dh])
    spk = _sc_edge_add(gtf, gge, tf2d, ge2d)
    pred4 = _tc4(spk, ef_pk, We1[2 * dh:], be1.reshape(1, dh),
                 We2, be2.reshape(1, 1))
    return pred4.reshape(-1)[:r]
